# baseline (speedup 1.0000x reference)
import functools

import jax
import jax.numpy as jnp
from jax import lax
from jax.experimental import pallas as pl
from jax.experimental.pallas import tpu as pltpu
from jax.experimental.pallas import tpu_sc as plsc

_NC = 2
_NS = 16
_NW = _NC * _NS

_L = 16
_CHUNK = 128


def _make_gather(S, B, E):
    tokens = S * B
    per_w = tokens // _NW
    bw = B // _NW
    assert bw == _CHUNK and per_w == S * _CHUNK

    mesh = plsc.VectorSubcoreMesh(core_axis_name="c", subcore_axis_name="s")

    @functools.partial(
        pl.kernel,
        mesh=mesh,
        compiler_params=pltpu.CompilerParams(needs_layout_passes=False),
        out_type=jax.ShapeDtypeStruct((S, E, B), jnp.float32),
        scratch_types=[
            pltpu.VMEM((per_w,), jnp.int32),
            pltpu.VMEM((per_w,), jnp.int32),
            pltpu.VMEM((per_w,), jnp.int32),
            pltpu.VMEM((_CHUNK, 2 * E), jnp.float32),
            pltpu.VMEM((E, _CHUNK), jnp.float32),
            pltpu.SemaphoreType.DMA,
        ],
    )
    def gather_kernel(table_hbm, idx_hbm, out_hbm, idx_v, pair_v, off_v,
                      buf_v, outb_v, sem):
        wid = lax.axis_index("s") * _NC + lax.axis_index("c")
        pltpu.sync_copy(idx_hbm.at[pl.ds(wid * per_w, per_w)], idx_v)

        def prep(k, carry):
            v = idx_v[pl.ds(k * _L, _L)]
            pair_v[pl.ds(k * _L, _L)] = lax.shift_right_logical(v, 1)
            off_v[pl.ds(k * _L, _L)] = lax.shift_left(
                lax.bitwise_and(v, 1), 6)
            return carry

        lax.fori_loop(0, per_w // _L, prep, 0)

        def chunk(s, carry):
            pltpu.async_copy(
                table_hbm.at[pair_v.at[pl.ds(s * _CHUNK, _CHUNK)]],
                buf_v, sem,
            ).wait()
            for tb in range(_CHUNK // _L):
                rows = lax.iota(jnp.int32, _L) + tb * _L
                offs = off_v[pl.ds(s * _CHUNK + tb * _L, _L)]

                def col(e, carry2):
                    outb_v[e, pl.ds(tb * _L, _L)] = plsc.load_gather(
                        buf_v, [rows, offs + e])
                    return carry2

                lax.fori_loop(0, E, col, 0)
            pltpu.sync_copy(outb_v,
                            out_hbm.at[s, :, pl.ds(wid * _CHUNK, _CHUNK)])
            return carry

        lax.fori_loop(0, S, chunk, 0)

    return gather_kernel


def kernel(input, table):
    seq, batch = input.shape
    vocab, embed = table.shape
    table2 = table.reshape(vocab // 2, 2 * embed)
    idxp = (input.reshape(seq, _NW, _CHUNK)
            .transpose(1, 0, 2)
            .reshape(-1))
    out = _make_gather(seq, batch, embed)(table2, idxp)
    return out.transpose(0, 2, 1)
